# baseline (speedup 1.0000x reference)
import functools

import jax
import jax.numpy as jnp
from jax import lax
from jax.experimental import pallas as pl
from jax.experimental.pallas import tpu as pltpu
from jax.experimental.pallas import tpu_sc as plsc

_HIDDEN = 64
_TABLE_ROWS = 100001
_BATCH = 16384

_info = plsc.get_sparse_core_info()
_NC, _NS = _info.num_cores, _info.num_subcores
_NW = _NC * _NS
_BPW = _BATCH // _NW

_mesh = plsc.VectorSubcoreMesh(core_axis_name="c", subcore_axis_name="s")


@functools.partial(
    pl.kernel,
    mesh=_mesh,
    out_type=jax.ShapeDtypeStruct((1, _BATCH, _HIDDEN), jnp.float32),
    scratch_types=[
        pltpu.VMEM((_BPW,), jnp.int32),
        pltpu.VMEM((_BPW, _HIDDEN), jnp.float32),
        pltpu.SemaphoreType.DMA((4,)),
        pltpu.SemaphoreType.DMA,
    ],
    compiler_params=pltpu.CompilerParams(disable_bounds_checks=True),
)
def _embed_gather(table_hbm, labels_hbm, out_hbm, idx_s, rows_v, sems, wsem):
    wid = lax.axis_index("s") * _NC + lax.axis_index("c")
    base = wid * _BPW
    pltpu.sync_copy(labels_hbm.at[pl.ds(base, _BPW)], idx_s)

    def fire_chunk(c):
        def fire(g, carry):
            b0 = c * 128 + g * 16
            vec = idx_s[pl.ds(b0, 16)]
            for k in range(16):
                r = vec[k]
                pltpu.async_copy(
                    table_hbm.at[0, pl.ds(r, 1)],
                    rows_v.at[pl.ds(b0 + k, 1)],
                    sems.at[c],
                )
            return carry

        lax.fori_loop(0, 8, fire, 0)

    def drain_and_write(c):
        pltpu.make_async_copy(
            table_hbm.at[0, pl.ds(0, 128)],
            rows_v.at[pl.ds(c * 128, 128)],
            sems.at[c],
        ).wait()
        return pltpu.async_copy(
            rows_v.at[pl.ds(c * 128, 128)],
            out_hbm.at[0, pl.ds(base + c * 128, 128)],
            wsem,
        )

    for c in range(4):
        fire_chunk(c)
    wcopies = [drain_and_write(c) for c in range(4)]
    for c in wcopies:
        c.wait()


def kernel(labels, embedding_table):
    table3 = embedding_table.reshape(1, _TABLE_ROWS, _HIDDEN)
    out3 = _embed_gather(table3, labels.astype(jnp.int32))
    return out3.reshape(_BATCH, _HIDDEN)


# speedup vs baseline: 1.0031x; 1.0031x over previous
import functools

import jax
import jax.numpy as jnp
from jax import lax
from jax.experimental import pallas as pl
from jax.experimental.pallas import tpu as pltpu
from jax.experimental.pallas import tpu_sc as plsc

_HIDDEN = 64
_TABLE_ROWS = 100001
_BATCH = 16384

_info = plsc.get_sparse_core_info()
_NC, _NS = _info.num_cores, _info.num_subcores
_NW = _NC * _NS
_BPW = _BATCH // _NW

_mesh = plsc.VectorSubcoreMesh(core_axis_name="c", subcore_axis_name="s")


@functools.partial(
    pl.kernel,
    mesh=_mesh,
    out_type=jax.ShapeDtypeStruct((_BATCH, _HIDDEN), jnp.float32),
    scratch_types=[
        pltpu.VMEM((_BPW,), jnp.int32),
        pltpu.VMEM((_BPW, _HIDDEN), jnp.float32),
        pltpu.SemaphoreType.DMA((4,)),
        pltpu.SemaphoreType.DMA,
    ],
    compiler_params=pltpu.CompilerParams(disable_bounds_checks=True),
)
def _embed_gather(table_hbm, labels_hbm, out_hbm, idx_s, rows_v, sems, wsem):
    wid = lax.axis_index("s") * _NC + lax.axis_index("c")
    base = wid * _BPW
    pltpu.sync_copy(labels_hbm.at[pl.ds(base, _BPW)], idx_s)

    def fire_chunk(c):
        def fire(g, carry):
            b0 = c * 128 + g * 16
            vec = idx_s[pl.ds(b0, 16)]
            for k in range(16):
                r = vec[k]
                pltpu.async_copy(
                    table_hbm.at[0, pl.ds(r, 1)],
                    rows_v.at[pl.ds(b0 + k, 1)],
                    sems.at[c],
                )
            return carry

        lax.fori_loop(0, 8, fire, 0)

    def drain_and_write(c):
        pltpu.make_async_copy(
            table_hbm.at[0, pl.ds(0, 128)],
            rows_v.at[pl.ds(c * 128, 128)],
            sems.at[c],
        ).wait()
        return pltpu.async_copy(
            rows_v.at[pl.ds(c * 128, 128)],
            out_hbm.at[pl.ds(base + c * 128, 128)],
            wsem,
        )

    for c in range(4):
        fire_chunk(c)
    wcopies = [drain_and_write(c) for c in range(4)]
    for c in wcopies:
        c.wait()


def kernel(labels, embedding_table):
    table3 = embedding_table.reshape(1, _TABLE_ROWS, _HIDDEN)
    return _embed_gather(table3, labels.astype(jnp.int32))
